# baseline (speedup 1.0000x reference)
import functools

import jax
import jax.numpy as jnp
from jax import lax
from jax.experimental import pallas as pl
from jax.experimental.pallas import tpu as pltpu
from jax.experimental.pallas import tpu_sc as plsc

_N = 100000
_E = 1600000
_DEPTH = 4
_NT = 16
_NODES_PT = 6272
_NP = _NT * _NODES_PT
_EPT = _E // _NT
_ECHUNK = 10000
_NCHUNK = _EPT // _ECHUNK
_HID = 64
_MLP_BL = 2560
_MLP_QL = 512


def _bf(v):
    return v.astype(jnp.bfloat16).astype(jnp.float32)


def _mlp_body(ea_ref, w1t_ref, b1_ref, w2_ref, b2_ref, out_ref):
    w1t = _bf(w1t_ref[...])
    w2 = _bf(w2_ref[...])
    for q in range(_MLP_BL // _MLP_QL):
        sl = slice(q * _MLP_QL, (q + 1) * _MLP_QL)
        ea = _bf(ea_ref[:, sl])
        h = (w1t[:, 0:1] * ea[0:1, :]
             + w1t[:, 1:2] * ea[1:2, :]
             + w1t[:, 2:3] * ea[2:3, :]) + b1_ref[...]
        h = _bf(jnp.maximum(h, 0.0))
        out_ref[:, sl] = (
            jnp.sum(h * w2, axis=0, keepdims=True) + b2_ref[...])


def _edge_weights(eaT, W1, b1, W2, b2):
    out = pl.pallas_call(
        _mlp_body,
        grid=(_E // _MLP_BL,),
        in_specs=[
            pl.BlockSpec((3, _MLP_BL), lambda i: (0, i)),
            pl.BlockSpec((_HID, 3), lambda i: (0, 0)),
            pl.BlockSpec((_HID, 1), lambda i: (0, 0)),
            pl.BlockSpec((_HID, 1), lambda i: (0, 0)),
            pl.BlockSpec((1, 1), lambda i: (0, 0)),
        ],
        out_specs=pl.BlockSpec((1, _MLP_BL), lambda i: (0, i)),
        out_shape=jax.ShapeDtypeStruct((1, _E), jnp.float32),
    )(eaT, W1.T, b1.reshape(_HID, 1), W2, b2.reshape(1, 1))
    return out.reshape(_E)


def _sc_message_passing(src, dst, w, x3p, r16, b16):
    mesh = plsc.VectorSubcoreMesh(
        core_axis_name="c", subcore_axis_name="s", num_cores=1)

    @functools.partial(
        pl.kernel,
        mesh=mesh,
        compiler_params=pltpu.CompilerParams(
            use_tc_tiling_on_sc=False, needs_layout_passes=False),
        out_type=jax.ShapeDtypeStruct((_NP,), jnp.float32),
        scratch_types=[
            pltpu.VMEM((_ECHUNK,), jnp.int32),
            pltpu.VMEM((_ECHUNK,), jnp.int32),
            pltpu.VMEM((_ECHUNK,), jnp.int32),
            pltpu.VMEM((_ECHUNK,), jnp.int32),
            pltpu.VMEM((_ECHUNK,), jnp.float32),
            pltpu.VMEM((_ECHUNK,), jnp.float32),
            pltpu.VMEM((_ECHUNK,), jnp.float32),
            pltpu.VMEM((_ECHUNK,), jnp.float32),
            pltpu.VMEM((_ECHUNK,), jnp.float32),
            pltpu.VMEM((_NODES_PT,), jnp.float32),
            pltpu.VMEM((_NODES_PT,), jnp.float32),
            pltpu.VMEM((_NODES_PT,), jnp.float32),
            pltpu.VMEM((16,), jnp.float32),
            pltpu.VMEM((16,), jnp.float32),
            pltpu.VMEM_SHARED((_NP,), jnp.float32),
            pltpu.VMEM_SHARED((_NP,), jnp.float32),
            pltpu.VMEM_SHARED((_NP,), jnp.float32),
            pltpu.SemaphoreType.DMA,
            pltpu.SemaphoreType.DMA,
            pltpu.SemaphoreType.DMA,
            pltpu.SemaphoreType.DMA,
            pltpu.SemaphoreType.DMA,
            pltpu.SemaphoreType.DMA,
            pltpu.SemaphoreType.DMA,
            pltpu.SemaphoreType.DMA,
            pltpu.SemaphoreType.DMA,
            pltpu.SemaphoreType.DMA,
            pltpu.SemaphoreType.DMA,
            pltpu.SemaphoreType.DMA,
        ],
    )
    def body(src_hbm, dst_hbm, w_hbm, x3_hbm, r_hbm, b_hbm, out_hbm,
             src0, src1, dst0, dst1, w0, w1, vals0, vals1, ones_v,
             acc_v, deg_v, xold_v, r_v, b_v,
             xk_sh, acc_sh, deg_sh,
             s_src0, s_src1, s_dst0, s_dst1, s_w0, s_w1,
             s_gat0, s_gat1, s_sct0, s_sct1, s_dg0, s_dg1):
        srcb = (src0, src1)
        dstb = (dst0, dst1)
        wb = (w0, w1)
        valsb = (vals0, vals1)
        s_src = (s_src0, s_src1)
        s_dst = (s_dst0, s_dst1)
        s_w = (s_w0, s_w1)
        s_gat = (s_gat0, s_gat1)
        s_sct = (s_sct0, s_sct1)
        s_dg = (s_dg0, s_dg1)

        tid = lax.axis_index("s")
        ebase = tid * _EPT
        nbase = tid * _NODES_PT
        nslice = pl.ds(nbase, _NODES_PT)

        pltpu.sync_copy(r_hbm, r_v)
        pltpu.sync_copy(b_hbm, b_v)
        root = r_v[...]
        bias = b_v[...]

        pltpu.sync_copy(x3_hbm.at[nslice], xk_sh.at[nslice])

        def fillz(i, _):
            xold_v[pl.ds(i * 16, 16)] = jnp.zeros((16,), jnp.float32)
            return 0
        lax.fori_loop(0, _NODES_PT // 16, fillz, 0)

        def fill1(i, _):
            ones_v[pl.ds(i * 16, 16)] = jnp.full((16,), 1.0, jnp.float32)
            return 0
        lax.fori_loop(0, _ECHUNK // 16, fill1, 0)

        pltpu.sync_copy(xold_v, acc_sh.at[nslice])
        pltpu.sync_copy(xold_v, deg_sh.at[nslice])
        plsc.subcore_barrier()

        def echunk(c):
            return pl.ds(ebase + c * _ECHUNK, _ECHUNK)

        def issue_in(c, b):
            return (
                pltpu.async_copy(src_hbm.at[echunk(c)], srcb[b], s_src[b]),
                pltpu.async_copy(w_hbm.at[echunk(c)], wb[b], s_w[b]),
                pltpu.async_copy(dst_hbm.at[echunk(c)], dstb[b], s_dst[b]),
            )

        def chunk_loop(count_deg):
            ins = [None] * _NCHUNK
            gats = [None] * _NCHUNK
            scts = [None] * _NCHUNK
            dscts = [None] * _NCHUNK
            ins[0] = issue_in(0, 0)
            ins[0][0].wait()
            gats[0] = pltpu.async_copy(xk_sh.at[src0], vals0, s_gat0)
            for c in range(_NCHUNK):
                b = c & 1
                if c >= 1:
                    scts[c - 1].wait()
                    if count_deg:
                        dscts[c - 1].wait()
                if c + 1 < _NCHUNK:
                    ins[c + 1] = issue_in(c + 1, 1 - b)
                gats[c].wait()
                ins[c][1].wait()

                def mul(i, _):
                    s = pl.ds(i * 16, 16)
                    valsb[b][s] = valsb[b][s] * wb[b][s]
                    return 0
                lax.fori_loop(0, _ECHUNK // 16, mul, 0)

                ins[c][2].wait()
                scts[c] = pltpu.async_copy(
                    valsb[b], acc_sh.at[dstb[b]], s_sct[b], add=True)
                if count_deg:
                    dscts[c] = pltpu.async_copy(
                        ones_v, deg_sh.at[dstb[b]], s_dg[b], add=True)
                if c + 1 < _NCHUNK:
                    ins[c + 1][0].wait()
                    gats[c + 1] = pltpu.async_copy(
                        xk_sh.at[srcb[1 - b]], valsb[1 - b], s_gat[1 - b])
            scts[_NCHUNK - 1].wait()
            if count_deg:
                dscts[_NCHUNK - 1].wait()
            plsc.subcore_barrier()

        def update():
            pltpu.sync_copy(acc_sh.at[nslice], acc_v)
            pltpu.sync_copy(xk_sh.at[nslice], xold_v)

            def upd(i, _):
                s = pl.ds(i * 16, 16)
                acc_v[s] = jnp.maximum(
                    acc_v[s] * deg_v[s] + xold_v[s] * root + bias, 0.0)
                return 0
            lax.fori_loop(0, _NODES_PT // 16, upd, 0)

            def zer(i, _):
                xold_v[pl.ds(i * 16, 16)] = jnp.zeros((16,), jnp.float32)
                return 0
            lax.fori_loop(0, _NODES_PT // 16, zer, 0)
            pltpu.sync_copy(xold_v, acc_sh.at[nslice])
            pltpu.sync_copy(acc_v, xk_sh.at[nslice])
            pltpu.sync_copy(acc_v, out_hbm.at[nslice])
            plsc.subcore_barrier()

        chunk_loop(count_deg=True)
        pltpu.sync_copy(deg_sh.at[nslice], deg_v)

        def inv(i, _):
            s = pl.ds(i * 16, 16)
            deg_v[s] = 1.0 / jnp.maximum(deg_v[s], 1.0)
            return 0
        lax.fori_loop(0, _NODES_PT // 16, inv, 0)
        update()

        def depth_body(t, _):
            chunk_loop(count_deg=False)
            update()
            return 0
        lax.fori_loop(1, _DEPTH, depth_body, 0)

    return body(src, dst, w, x3p, r16, b16)


def kernel(x, edge_index, edge_attr, W1, b1, W2, b2, root, bias):
    w = _edge_weights(edge_attr.T, W1, b1, W2, b2)
    x3p = jnp.pad(x[:, 2], (0, _NP - _N))
    r16 = jnp.full((16,), root[0, 0], jnp.float32)
    b16 = jnp.full((16,), bias[0], jnp.float32)
    out = _sc_message_passing(edge_index[0], edge_index[1], w, x3p, r16, b16)
    return out[:_N].reshape(_N, 1)


# speedup vs baseline: 1.3091x; 1.3091x over previous
import functools

import jax
import jax.numpy as jnp
from jax import lax
from jax.experimental import pallas as pl
from jax.experimental.pallas import tpu as pltpu
from jax.experimental.pallas import tpu_sc as plsc

_N = 100000
_E = 1600000
_DEPTH = 4
_NT = 16
_NODES_PT = 6272
_NP = _NT * _NODES_PT
_EPT = _E // _NT
_ECHUNK = 10000
_NCHUNK = _EPT // _ECHUNK
_HID = 64
_MLP_BL = 12800
_MLP_QL = 512


def _bf(v):
    return v.astype(jnp.bfloat16).astype(jnp.float32)


def _mlp_body(ea_ref, w1t_ref, b1_ref, w2_ref, b2_ref, out_ref):
    w1t = _bf(w1t_ref[...])
    w2 = _bf(w2_ref[...])
    for q in range(_MLP_BL // _MLP_QL):
        sl = slice(q * _MLP_QL, (q + 1) * _MLP_QL)
        ea = _bf(ea_ref[:, sl])
        h = (w1t[:, 0:1] * ea[0:1, :]
             + w1t[:, 1:2] * ea[1:2, :]
             + w1t[:, 2:3] * ea[2:3, :]) + b1_ref[...]
        h = _bf(jnp.maximum(h, 0.0))
        out_ref[:, sl] = (
            jnp.sum(h * w2, axis=0, keepdims=True) + b2_ref[...])


def _edge_weights(eaT, W1, b1, W2, b2):
    out = pl.pallas_call(
        _mlp_body,
        grid=(_E // _MLP_BL,),
        in_specs=[
            pl.BlockSpec((3, _MLP_BL), lambda i: (0, i)),
            pl.BlockSpec((_HID, 3), lambda i: (0, 0)),
            pl.BlockSpec((_HID, 1), lambda i: (0, 0)),
            pl.BlockSpec((_HID, 1), lambda i: (0, 0)),
            pl.BlockSpec((1, 1), lambda i: (0, 0)),
        ],
        out_specs=pl.BlockSpec((1, _MLP_BL), lambda i: (0, i)),
        out_shape=jax.ShapeDtypeStruct((1, _E), jnp.float32),
    )(eaT, W1.T, b1.reshape(_HID, 1), W2, b2.reshape(1, 1))
    return out.reshape(_E)


def _sc_message_passing(ei, w, x3p, r16, b16):
    mesh = plsc.VectorSubcoreMesh(
        core_axis_name="c", subcore_axis_name="s", num_cores=1)

    @functools.partial(
        pl.kernel,
        mesh=mesh,
        compiler_params=pltpu.CompilerParams(
            use_tc_tiling_on_sc=False, needs_layout_passes=False),
        out_type=jax.ShapeDtypeStruct((_NP,), jnp.float32),
        scratch_types=[
            pltpu.VMEM((_ECHUNK,), jnp.int32),
            pltpu.VMEM((_ECHUNK,), jnp.int32),
            pltpu.VMEM((_ECHUNK,), jnp.int32),
            pltpu.VMEM((_ECHUNK,), jnp.int32),
            pltpu.VMEM((_ECHUNK,), jnp.float32),
            pltpu.VMEM((_ECHUNK,), jnp.float32),
            pltpu.VMEM((_ECHUNK,), jnp.float32),
            pltpu.VMEM((_ECHUNK,), jnp.float32),
            pltpu.VMEM((_ECHUNK,), jnp.float32),
            pltpu.VMEM((_NODES_PT,), jnp.float32),
            pltpu.VMEM((_NODES_PT,), jnp.float32),
            pltpu.VMEM((_NODES_PT,), jnp.float32),
            pltpu.VMEM((16,), jnp.float32),
            pltpu.VMEM((16,), jnp.float32),
            pltpu.VMEM_SHARED((_NP,), jnp.float32),
            pltpu.VMEM_SHARED((_NP,), jnp.float32),
            pltpu.VMEM_SHARED((_NP,), jnp.float32),
            pltpu.SemaphoreType.DMA,
            pltpu.SemaphoreType.DMA,
            pltpu.SemaphoreType.DMA,
            pltpu.SemaphoreType.DMA,
            pltpu.SemaphoreType.DMA,
            pltpu.SemaphoreType.DMA,
            pltpu.SemaphoreType.DMA,
            pltpu.SemaphoreType.DMA,
            pltpu.SemaphoreType.DMA,
            pltpu.SemaphoreType.DMA,
            pltpu.SemaphoreType.DMA,
            pltpu.SemaphoreType.DMA,
        ],
    )
    def body(ei_hbm, w_hbm, x3_hbm, r_hbm, b_hbm, out_hbm,
             src0, src1, dst0, dst1, w0, w1, vals0, vals1, ones_v,
             acc_v, deg_v, xold_v, r_v, b_v,
             xk_sh, acc_sh, deg_sh,
             s_src0, s_src1, s_dst0, s_dst1, s_w0, s_w1,
             s_gat0, s_gat1, s_sct0, s_sct1, s_dg0, s_dg1):
        srcb = (src0, src1)
        dstb = (dst0, dst1)
        wb = (w0, w1)
        valsb = (vals0, vals1)
        s_src = (s_src0, s_src1)
        s_dst = (s_dst0, s_dst1)
        s_w = (s_w0, s_w1)
        s_gat = (s_gat0, s_gat1)
        s_sct = (s_sct0, s_sct1)
        s_dg = (s_dg0, s_dg1)

        tid = lax.axis_index("s")
        ebase = tid * _EPT
        nbase = tid * _NODES_PT
        nslice = pl.ds(nbase, _NODES_PT)

        pltpu.sync_copy(r_hbm, r_v)
        pltpu.sync_copy(b_hbm, b_v)
        root = r_v[...]
        bias = b_v[...]

        pltpu.sync_copy(x3_hbm.at[nslice], xk_sh.at[nslice])

        def fillz(i, _):
            xold_v[pl.ds(i * 16, 16)] = jnp.zeros((16,), jnp.float32)
            return 0
        lax.fori_loop(0, _NODES_PT // 16, fillz, 0)

        def fill1(i, _):
            ones_v[pl.ds(i * 16, 16)] = jnp.full((16,), 1.0, jnp.float32)
            return 0
        lax.fori_loop(0, _ECHUNK // 16, fill1, 0)

        pltpu.sync_copy(xold_v, acc_sh.at[nslice])
        pltpu.sync_copy(xold_v, deg_sh.at[nslice])
        plsc.subcore_barrier()

        def echunk(c):
            return pl.ds(ebase + c * _ECHUNK, _ECHUNK)

        def issue_in(c, b):
            return (
                pltpu.async_copy(ei_hbm.at[0, echunk(c)], srcb[b], s_src[b]),
                pltpu.async_copy(w_hbm.at[echunk(c)], wb[b], s_w[b]),
                pltpu.async_copy(ei_hbm.at[1, echunk(c)], dstb[b], s_dst[b]),
            )

        def chunk_loop(count_deg):
            ins = [None] * _NCHUNK
            gats = [None] * _NCHUNK
            scts = [None] * _NCHUNK
            dscts = [None] * _NCHUNK
            ins[0] = issue_in(0, 0)
            ins[0][0].wait()
            gats[0] = pltpu.async_copy(xk_sh.at[src0], vals0, s_gat0)
            for c in range(_NCHUNK):
                b = c & 1
                if c >= 1:
                    scts[c - 1].wait()
                    if count_deg:
                        dscts[c - 1].wait()
                if c + 1 < _NCHUNK:
                    ins[c + 1] = issue_in(c + 1, 1 - b)
                gats[c].wait()
                ins[c][1].wait()

                def mul(i, _):
                    s = pl.ds(i * 16, 16)
                    valsb[b][s] = valsb[b][s] * wb[b][s]
                    return 0
                lax.fori_loop(0, _ECHUNK // 16, mul, 0)

                ins[c][2].wait()
                scts[c] = pltpu.async_copy(
                    valsb[b], acc_sh.at[dstb[b]], s_sct[b], add=True)
                if count_deg:
                    dscts[c] = pltpu.async_copy(
                        ones_v, deg_sh.at[dstb[b]], s_dg[b], add=True)
                if c + 1 < _NCHUNK:
                    ins[c + 1][0].wait()
                    gats[c + 1] = pltpu.async_copy(
                        xk_sh.at[srcb[1 - b]], valsb[1 - b], s_gat[1 - b])
            scts[_NCHUNK - 1].wait()
            if count_deg:
                dscts[_NCHUNK - 1].wait()
            plsc.subcore_barrier()

        def update():
            pltpu.sync_copy(acc_sh.at[nslice], acc_v)
            pltpu.sync_copy(xk_sh.at[nslice], xold_v)

            def upd(i, _):
                s = pl.ds(i * 16, 16)
                acc_v[s] = jnp.maximum(
                    acc_v[s] * deg_v[s] + xold_v[s] * root + bias, 0.0)
                return 0
            lax.fori_loop(0, _NODES_PT // 16, upd, 0)

            def zer(i, _):
                xold_v[pl.ds(i * 16, 16)] = jnp.zeros((16,), jnp.float32)
                return 0
            lax.fori_loop(0, _NODES_PT // 16, zer, 0)
            pltpu.sync_copy(xold_v, acc_sh.at[nslice])
            pltpu.sync_copy(acc_v, xk_sh.at[nslice])
            pltpu.sync_copy(acc_v, out_hbm.at[nslice])
            plsc.subcore_barrier()

        chunk_loop(count_deg=True)
        pltpu.sync_copy(deg_sh.at[nslice], deg_v)

        def inv(i, _):
            s = pl.ds(i * 16, 16)
            deg_v[s] = 1.0 / jnp.maximum(deg_v[s], 1.0)
            return 0
        lax.fori_loop(0, _NODES_PT // 16, inv, 0)
        update()

        def depth_body(t, _):
            chunk_loop(count_deg=False)
            update()
            return 0
        lax.fori_loop(1, _DEPTH, depth_body, 0)

    return body(ei, w, x3p, r16, b16)


def kernel(x, edge_index, edge_attr, W1, b1, W2, b2, root, bias):
    w = _edge_weights(edge_attr.T, W1, b1, W2, b2)
    x3p = jnp.pad(x[:, 2], (0, _NP - _N))
    r16 = jnp.full((16,), root[0, 0], jnp.float32)
    b16 = jnp.full((16,), bias[0], jnp.float32)
    out = _sc_message_passing(edge_index, w, x3p, r16, b16)
    return out[:_N].reshape(_N, 1)
